# baseline (speedup 1.0000x reference)
import functools

import jax
import jax.numpy as jnp
from jax import lax
from jax.experimental import pallas as pl
from jax.experimental.pallas import tpu as pltpu
from jax.experimental.pallas import tpu_sc as plsc

_NC = 2
_NS = 16
_NW = _NC * _NS
_L = 16


@functools.lru_cache(maxsize=None)
def _make_gather2(B, D, V):
  b_per_w = B // _NW
  G = b_per_w // _L
  mesh = plsc.VectorSubcoreMesh(core_axis_name="c", subcore_axis_name="s")

  @functools.partial(
      pl.kernel,
      mesh=mesh,
      out_type=(
          jax.ShapeDtypeStruct((B, D), jnp.float32),
          jax.ShapeDtypeStruct((B, D), jnp.float32),
      ),
      compiler_params=pltpu.CompilerParams(needs_layout_passes=False),
      scratch_types=[
          pltpu.VMEM((b_per_w,), jnp.int32),
          pltpu.VMEM((4, D, 128), jnp.float32),
          pltpu.VMEM((b_per_w, D), jnp.float32),
          pltpu.SemaphoreType.DMA,
      ],
  )
  def gather2(uid_hbm, iid_hbm, utT_hbm, itT_hbm, ue_hbm, ie_hbm,
              idx_v, blocks_v, rows_v, sem):
    wid = lax.axis_index("s") * _NC + lax.axis_index("c")
    base = wid * b_per_w
    lanes = lax.iota(jnp.int32, _L)

    def one_table(ids_hbm, tT_hbm, out_hbm):
      pltpu.sync_copy(ids_hbm.at[pl.ds(base, b_per_w)], idx_v)

      def group(g, carry):
        vec = idx_v[pl.ds(g * _L, _L)]
        p0 = g * _L
        for h in range(4):
          vjs = []
          copies = []
          for j in range(4):
            vj = lax.reduce_max(jnp.where(lanes == h * 4 + j, vec, 0), (0,))
            c0 = pl.multiple_of((vj // 128) * 128, 128)
            vjs.append(vj)
            copies.append(pltpu.async_copy(
                tT_hbm.at[:, pl.ds(c0, 128)], blocks_v.at[j], sem))
          for j in range(4):
            copies[j].wait()
            lvec = jnp.full((_L,), vjs[j] % 128, jnp.int32)
            rowvec = jnp.full((_L,), p0 + h * 4 + j, jnp.int32)
            for q in range(D // _L):
              dvec = lanes + q * _L
              vals = plsc.load_gather(blocks_v.at[j], [dvec, lvec])
              plsc.store_scatter(rows_v, [rowvec, dvec], vals)
        return carry

      lax.fori_loop(0, G, group, 0)
      pltpu.sync_copy(rows_v, out_hbm.at[pl.ds(base, b_per_w)])

    one_table(uid_hbm, utT_hbm, ue_hbm)
    one_table(iid_hbm, itT_hbm, ie_hbm)

  return gather2


_K = 16


@functools.lru_cache(maxsize=None)
def _make_tc_gather(Btc, D, V):
  nsteps = Btc // _K

  def body(ids_ref, *refs):
    out_ref = refs[_K]
    i = pl.program_id(0)
    lanes = jax.lax.broadcasted_iota(jnp.int32, (1, 128), 1)
    for k in range(_K):
      l = ids_ref[i * _K + k] % 128
      oh = (lanes == l).astype(jnp.float32)
      col = jax.lax.dot_general(oh, refs[k][...], (((1,), (1,)), ((), ())),
                                preferred_element_type=jnp.float32)
      out_ref[k, :] = col[0, :]

  in_specs = [
      pl.BlockSpec((D, 128), functools.partial(
          lambda k, i, ids: (0, ids[i * _K + k] // 128), k))
      for k in range(_K)
  ]
  grid_spec = pltpu.PrefetchScalarGridSpec(
      num_scalar_prefetch=1,
      grid=(nsteps,),
      in_specs=in_specs,
      out_specs=pl.BlockSpec((_K, D), lambda i, ids: (i, 0)),
  )
  return pl.pallas_call(
      body,
      grid_spec=grid_spec,
      out_shape=jax.ShapeDtypeStruct((Btc, D), jnp.float32),
  )


def _mlp_body(ue, ie, w1u, w1i, b1, w2, b2, w3, b3, wo, bo, out):
  x = jnp.dot(ue[...], w1u[...], preferred_element_type=jnp.float32)
  x = x + jnp.dot(ie[...], w1i[...], preferred_element_type=jnp.float32)
  x = jnp.maximum(x + b1[...], 0.0)
  x = jnp.maximum(
      jnp.dot(x, w2[...], preferred_element_type=jnp.float32) + b2[...], 0.0)
  x = jnp.maximum(
      jnp.dot(x, w3[...], preferred_element_type=jnp.float32) + b3[...], 0.0)
  out[...] = jnp.dot(x, wo[...], preferred_element_type=jnp.float32) + bo[...]


def kernel(user_ids, item_ids, user_table, item_table,
           W1, b1, W2, b2, W3, b3, Wout, bout):
  B = user_ids.shape[0]
  V = user_table.shape[0]
  D = user_table.shape[1]
  H1, H2, H3 = W1.shape[0], W2.shape[0], W3.shape[0]

  Bsc = 6144
  Btc = B - Bsc
  uids = user_ids.astype(jnp.int32)
  iids = item_ids.astype(jnp.int32)
  utT = user_table.T
  itT = item_table.T

  ue_sc, ie_sc = _make_gather2(Bsc, D, V)(
      uids[:Bsc], iids[:Bsc], utT, itT)
  tc_gather = _make_tc_gather(Btc, D, V)
  ue_tc = tc_gather(uids[Bsc:], *([utT] * _K))
  ie_tc = tc_gather(iids[Bsc:], *([itT] * _K))
  ue = jnp.concatenate([ue_sc, ue_tc], axis=0)
  ie = jnp.concatenate([ie_sc, ie_tc], axis=0)

  w1u = W1[:, :D].T
  w1i = W1[:, D:].T
  w2t = W2.T
  w3t = W3.T
  wot = Wout.T
  b1r = b1.reshape(1, H1)
  b2r = b2.reshape(1, H2)
  b3r = b3.reshape(1, H3)
  bor = bout.reshape(1, 1)

  blk = 2048
  full = lambda r, c: pl.BlockSpec((r, c), lambda i: (0, 0))
  out = pl.pallas_call(
      _mlp_body,
      grid=(B // blk,),
      in_specs=[
          pl.BlockSpec((blk, D), lambda i: (i, 0)),
          pl.BlockSpec((blk, D), lambda i: (i, 0)),
          full(D, H1), full(D, H1), full(1, H1),
          full(H1, H2), full(1, H2),
          full(H2, H3), full(1, H3),
          full(H3, 1), full(1, 1),
      ],
      out_specs=pl.BlockSpec((blk, 1), lambda i: (i, 0)),
      out_shape=jax.ShapeDtypeStruct((B, 1), jnp.float32),
  )(ue, ie, w1u, w1i, b1r, w2t, b2r, w3t, b3r, wot, bor)
  return out


# speedup vs baseline: 1.8126x; 1.8126x over previous
import functools

import jax
import jax.numpy as jnp
from jax import lax
from jax.experimental import pallas as pl
from jax.experimental.pallas import tpu as pltpu
from jax.experimental.pallas import tpu_sc as plsc

_NC = 2
_NS = 16
_NW = _NC * _NS
_L = 16


@functools.lru_cache(maxsize=None)
def _make_gather2(B, D, V):
  b_per_w = B // _NW
  G = b_per_w // _L
  mesh = plsc.VectorSubcoreMesh(core_axis_name="c", subcore_axis_name="s")

  @functools.partial(
      pl.kernel,
      mesh=mesh,
      out_type=(
          jax.ShapeDtypeStruct((B, D), jnp.float32),
          jax.ShapeDtypeStruct((B, D), jnp.float32),
      ),
      compiler_params=pltpu.CompilerParams(needs_layout_passes=False),
      scratch_types=[
          pltpu.VMEM((b_per_w,), jnp.int32),
          pltpu.VMEM((4, D, 128), jnp.float32),
          pltpu.VMEM((b_per_w, D), jnp.float32),
          pltpu.SemaphoreType.DMA,
      ],
  )
  def gather2(uid_hbm, iid_hbm, utT_hbm, itT_hbm, ue_hbm, ie_hbm,
              idx_v, blocks_v, rows_v, sem):
    wid = lax.axis_index("s") * _NC + lax.axis_index("c")
    base = wid * b_per_w
    lanes = lax.iota(jnp.int32, _L)

    def one_table(ids_hbm, tT_hbm, out_hbm):
      pltpu.sync_copy(ids_hbm.at[pl.ds(base, b_per_w)], idx_v)

      def group(g, carry):
        vec = idx_v[pl.ds(g * _L, _L)]
        p0 = g * _L
        for h in range(4):
          vjs = []
          copies = []
          for j in range(4):
            vj = lax.reduce_max(jnp.where(lanes == h * 4 + j, vec, 0), (0,))
            c0 = pl.multiple_of((vj // 128) * 128, 128)
            vjs.append(vj)
            copies.append(pltpu.async_copy(
                tT_hbm.at[:, pl.ds(c0, 128)], blocks_v.at[j], sem))
          for j in range(4):
            copies[j].wait()
            lvec = jnp.full((_L,), vjs[j] % 128, jnp.int32)
            rowvec = jnp.full((_L,), p0 + h * 4 + j, jnp.int32)
            for q in range(D // _L):
              dvec = lanes + q * _L
              vals = plsc.load_gather(blocks_v.at[j], [dvec, lvec])
              plsc.store_scatter(rows_v, [rowvec, dvec], vals)
        return carry

      lax.fori_loop(0, G, group, 0)
      pltpu.sync_copy(rows_v, out_hbm.at[pl.ds(base, b_per_w)])

    one_table(uid_hbm, utT_hbm, ue_hbm)
    one_table(iid_hbm, itT_hbm, ie_hbm)

  return gather2


_K = 32


@functools.lru_cache(maxsize=None)
def _make_tc_gather(Btc, D, V):
  nsteps = Btc // _K

  def body(ids_ref, idv_ref, *refs):
    out_ref = refs[_K]
    idv = idv_ref[0, 0, :]
    lanes = jax.lax.broadcasted_iota(jnp.int32, (_K, 128), 1)
    oh = (lanes == (idv % 128)[:, None]).astype(jnp.float32)
    for k in range(_K):
      col = jax.lax.dot_general(
          oh[k:k + 1, :], refs[k][...], (((1,), (1,)), ((), ())),
          preferred_element_type=jnp.float32)
      out_ref[k, :] = col[0, :]

  in_specs = [pl.BlockSpec((1, 1, _K), lambda i, ids: (i, 0, 0))] + [
      pl.BlockSpec((D, 128), functools.partial(
          lambda k, i, ids: (0, ids[i * _K + k] // 128), k))
      for k in range(_K)
  ]
  grid_spec = pltpu.PrefetchScalarGridSpec(
      num_scalar_prefetch=1,
      grid=(nsteps,),
      in_specs=in_specs,
      out_specs=pl.BlockSpec((_K, D), lambda i, ids: (i, 0)),
  )
  return pl.pallas_call(
      body,
      grid_spec=grid_spec,
      out_shape=jax.ShapeDtypeStruct((Btc, D), jnp.float32),
  )


def _mlp_body(ue, ie, w1u, w1i, b1, w2, b2, w3, b3, wo, bo, out):
  x = jnp.dot(ue[...], w1u[...], preferred_element_type=jnp.float32)
  x = x + jnp.dot(ie[...], w1i[...], preferred_element_type=jnp.float32)
  x = jnp.maximum(x + b1[...], 0.0)
  x = jnp.maximum(
      jnp.dot(x, w2[...], preferred_element_type=jnp.float32) + b2[...], 0.0)
  x = jnp.maximum(
      jnp.dot(x, w3[...], preferred_element_type=jnp.float32) + b3[...], 0.0)
  out[...] = jnp.dot(x, wo[...], preferred_element_type=jnp.float32) + bo[...]


def kernel(user_ids, item_ids, user_table, item_table,
           W1, b1, W2, b2, W3, b3, Wout, bout):
  B = user_ids.shape[0]
  V = user_table.shape[0]
  D = user_table.shape[1]
  H1, H2, H3 = W1.shape[0], W2.shape[0], W3.shape[0]

  Bsc = 10240
  Btc = B - Bsc
  uids = user_ids.astype(jnp.int32)
  iids = item_ids.astype(jnp.int32)
  utT = user_table.T
  itT = item_table.T

  ue_sc, ie_sc = _make_gather2(Bsc, D, V)(
      uids[:Bsc], iids[:Bsc], utT, itT)
  tc_gather = _make_tc_gather(Btc, D, V)
  uids3 = uids[Bsc:].reshape(Btc // _K, 1, _K)
  iids3 = iids[Bsc:].reshape(Btc // _K, 1, _K)
  ue_tc = tc_gather(uids[Bsc:], uids3, *([utT] * _K))
  ie_tc = tc_gather(iids[Bsc:], iids3, *([itT] * _K))
  ue = jnp.concatenate([ue_sc, ue_tc], axis=0)
  ie = jnp.concatenate([ie_sc, ie_tc], axis=0)

  w1u = W1[:, :D].T
  w1i = W1[:, D:].T
  w2t = W2.T
  w3t = W3.T
  wot = Wout.T
  b1r = b1.reshape(1, H1)
  b2r = b2.reshape(1, H2)
  b3r = b3.reshape(1, H3)
  bor = bout.reshape(1, 1)

  blk = 2048
  full = lambda r, c: pl.BlockSpec((r, c), lambda i: (0, 0))
  out = pl.pallas_call(
      _mlp_body,
      grid=(B // blk,),
      in_specs=[
          pl.BlockSpec((blk, D), lambda i: (i, 0)),
          pl.BlockSpec((blk, D), lambda i: (i, 0)),
          full(D, H1), full(D, H1), full(1, H1),
          full(H1, H2), full(1, H2),
          full(H2, H3), full(1, H3),
          full(H3, 1), full(1, 1),
      ],
      out_specs=pl.BlockSpec((blk, 1), lambda i: (i, 0)),
      out_shape=jax.ShapeDtypeStruct((B, 1), jnp.float32),
  )(ue, ie, w1u, w1i, b1r, w2t, b2r, w3t, b3r, wot, bor)
  return out


# speedup vs baseline: 2.3449x; 1.2936x over previous
import functools

import jax
import jax.numpy as jnp
from jax import lax
from jax.experimental import pallas as pl
from jax.experimental.pallas import tpu as pltpu
from jax.experimental.pallas import tpu_sc as plsc

_NC = 2
_NS = 16
_NW = _NC * _NS
_L = 16


@functools.lru_cache(maxsize=None)
def _make_gather2(B, D, V):
  b_per_w = B // _NW
  G = b_per_w // _L
  mesh = plsc.VectorSubcoreMesh(core_axis_name="c", subcore_axis_name="s")

  @functools.partial(
      pl.kernel,
      mesh=mesh,
      out_type=(
          jax.ShapeDtypeStruct((B, D), jnp.float32),
          jax.ShapeDtypeStruct((B, D), jnp.float32),
      ),
      compiler_params=pltpu.CompilerParams(needs_layout_passes=False),
      scratch_types=[
          pltpu.VMEM((b_per_w,), jnp.int32),
          pltpu.VMEM((4, D, 128), jnp.float32),
          pltpu.VMEM((b_per_w, D), jnp.float32),
          pltpu.SemaphoreType.DMA,
      ],
  )
  def gather2(uid_hbm, iid_hbm, utT_hbm, itT_hbm, ue_hbm, ie_hbm,
              idx_v, blocks_v, rows_v, sem):
    wid = lax.axis_index("s") * _NC + lax.axis_index("c")
    base = wid * b_per_w
    lanes = lax.iota(jnp.int32, _L)

    def one_table(ids_hbm, tT_hbm, out_hbm):
      pltpu.sync_copy(ids_hbm.at[pl.ds(base, b_per_w)], idx_v)

      def group(g, carry):
        vec = idx_v[pl.ds(g * _L, _L)]
        p0 = g * _L
        for h in range(4):
          vjs = []
          copies = []
          for j in range(4):
            vj = lax.reduce_max(jnp.where(lanes == h * 4 + j, vec, 0), (0,))
            c0 = pl.multiple_of((vj // 128) * 128, 128)
            vjs.append(vj)
            copies.append(pltpu.async_copy(
                tT_hbm.at[:, pl.ds(c0, 128)], blocks_v.at[j], sem))
          for j in range(4):
            copies[j].wait()
            lvec = jnp.full((_L,), vjs[j] % 128, jnp.int32)
            rowvec = jnp.full((_L,), p0 + h * 4 + j, jnp.int32)
            for q in range(D // _L):
              dvec = lanes + q * _L
              vals = plsc.load_gather(blocks_v.at[j], [dvec, lvec])
              plsc.store_scatter(rows_v, [rowvec, dvec], vals)
        return carry

      lax.fori_loop(0, G, group, 0)
      pltpu.sync_copy(rows_v, out_hbm.at[pl.ds(base, b_per_w)])

    one_table(uid_hbm, utT_hbm, ue_hbm)
    one_table(iid_hbm, itT_hbm, ie_hbm)

  return gather2


_K = 32


@functools.lru_cache(maxsize=None)
def _make_tc_gather(Btc, D, V):
  nsteps = Btc // _K

  def body(ids_ref, idv_ref, *refs):
    out_ref = refs[_K]
    idv = idv_ref[0, 0, :]
    lanes = jax.lax.broadcasted_iota(jnp.int32, (_K, 128), 1)
    oh = (lanes == (idv % 128)[:, None]).astype(jnp.float32)
    for k in range(_K):
      col = jax.lax.dot_general(
          oh[k:k + 1, :], refs[k][...], (((1,), (1,)), ((), ())),
          preferred_element_type=jnp.float32)
      out_ref[k, :] = col[0, :]

  in_specs = [pl.BlockSpec((1, 1, _K), lambda i, ids: (i, 0, 0))] + [
      pl.BlockSpec((D, 128), functools.partial(
          lambda k, i, ids: (0, ids[i * _K + k] // 128), k))
      for k in range(_K)
  ]
  grid_spec = pltpu.PrefetchScalarGridSpec(
      num_scalar_prefetch=1,
      grid=(nsteps,),
      in_specs=in_specs,
      out_specs=pl.BlockSpec((_K, D), lambda i, ids: (i, 0)),
  )
  return pl.pallas_call(
      body,
      grid_spec=grid_spec,
      out_shape=jax.ShapeDtypeStruct((Btc, D), jnp.float32),
  )


def _mlp_body(ue, ie, w1u, w1i, b1, w2, b2, w3, b3, wo, bo, out):
  x = jnp.dot(ue[...], w1u[...], preferred_element_type=jnp.float32)
  x = x + jnp.dot(ie[...], w1i[...], preferred_element_type=jnp.float32)
  x = jnp.maximum(x + b1[...], 0.0)
  x = jnp.maximum(
      jnp.dot(x, w2[...], preferred_element_type=jnp.float32) + b2[...], 0.0)
  x = jnp.maximum(
      jnp.dot(x, w3[...], preferred_element_type=jnp.float32) + b3[...], 0.0)
  out[...] = jnp.dot(x, wo[...], preferred_element_type=jnp.float32) + bo[...]


def kernel(user_ids, item_ids, user_table, item_table,
           W1, b1, W2, b2, W3, b3, Wout, bout):
  B = user_ids.shape[0]
  V = user_table.shape[0]
  D = user_table.shape[1]
  H1, H2, H3 = W1.shape[0], W2.shape[0], W3.shape[0]

  Bsc = 11776
  Btc = B - Bsc
  uids = user_ids.astype(jnp.int32)
  iids = item_ids.astype(jnp.int32)
  utT = user_table.T
  itT = item_table.T

  ue_sc, ie_sc = _make_gather2(Bsc, D, V)(
      uids[:Bsc], iids[:Bsc], utT, itT)
  tc_gather = _make_tc_gather(Btc, D, V)
  uids3 = uids[Bsc:].reshape(Btc // _K, 1, _K)
  iids3 = iids[Bsc:].reshape(Btc // _K, 1, _K)
  ue_tc = tc_gather(uids[Bsc:], uids3, *([utT] * _K))
  ie_tc = tc_gather(iids[Bsc:], iids3, *([itT] * _K))
  ue = jnp.concatenate([ue_sc, ue_tc], axis=0)
  ie = jnp.concatenate([ie_sc, ie_tc], axis=0)

  w1u = W1[:, :D].T
  w1i = W1[:, D:].T
  w2t = W2.T
  w3t = W3.T
  wot = Wout.T
  b1r = b1.reshape(1, H1)
  b2r = b2.reshape(1, H2)
  b3r = b3.reshape(1, H3)
  bor = bout.reshape(1, 1)

  blk = 2048
  full = lambda r, c: pl.BlockSpec((r, c), lambda i: (0, 0))
  out = pl.pallas_call(
      _mlp_body,
      grid=(B // blk,),
      in_specs=[
          pl.BlockSpec((blk, D), lambda i: (i, 0)),
          pl.BlockSpec((blk, D), lambda i: (i, 0)),
          full(D, H1), full(D, H1), full(1, H1),
          full(H1, H2), full(1, H2),
          full(H2, H3), full(1, H3),
          full(H3, 1), full(1, 1),
      ],
      out_specs=pl.BlockSpec((blk, 1), lambda i: (i, 0)),
      out_shape=jax.ShapeDtypeStruct((B, 1), jnp.float32),
  )(ue, ie, w1u, w1i, b1r, w2t, b2r, w3t, b3r, wot, bor)
  return out
